# Initial kernel scaffold; baseline (speedup 1.0000x reference)
import jax
import jax.numpy as jnp
from jax.experimental import pallas as pl


def kernel(x, W, b, mem_keys, mem_values):
    raise NotImplementedError("write your pallas kernel here")


# speedup vs baseline: 1.4184x; 1.4184x over previous
import functools

import jax
import jax.numpy as jnp
from jax import lax
from jax.experimental import pallas as pl
from jax.experimental.pallas import tpu as pltpu
from jax.experimental.pallas import tpu_sc as plsc

MEM = 1000000
D = 32
B = 32
K = 32
EPS = 1e-8
CHUNK = 8192
NEG = -3.0


def _scan_body(qn_ref, keys_ref, d_ref, idx_out, wt_out, rv_ref, ri_ref):
    i = pl.program_id(0)
    nb = pl.num_programs(0)

    @pl.when(i == 0)
    def _init():
        rv_ref[...] = jnp.full((B, K), NEG, jnp.float32)
        ri_ref[...] = jnp.zeros((B, K), jnp.int32)

    kn = keys_ref[...] / d_ref[...]
    s_raw = lax.dot_general(qn_ref[...], kn, (((1,), (1,)), ((), ())),
                            preferred_element_type=jnp.float32)
    lane = lax.broadcasted_iota(jnp.int32, (1, CHUNK), 1)
    valid = (i * CHUNK + lane) < MEM
    s = jnp.where(valid, s_raw, NEG)

    def calc_cand(m, im):
        after = (s < m) | ((s == m) & (lane > im))
        sm = jnp.where(after, s, NEG)
        cm = jnp.max(sm, axis=1, keepdims=True)
        ci = jnp.min(jnp.where(sm == cm, lane, CHUNK), axis=1, keepdims=True)
        return cm, ci

    kpos = lax.broadcasted_iota(jnp.int32, (B, K), 1)
    rv0 = rv_ref[...]
    ri0 = ri_ref[...]
    cm0, ci0 = calc_cand(jnp.full((B, 1), jnp.inf, jnp.float32),
                         jnp.full((B, 1), -1, jnp.int32))

    def cond(carry):
        cm, _, rv, _ = carry
        return jnp.any(cm > rv[:, K - 1:K])

    def body(carry):
        cm, ci, rv, ri = carry
        gi = i * CHUNK + ci
        do = cm > rv[:, K - 1:K]
        p = jnp.sum((rv >= cm).astype(jnp.int32), axis=1, keepdims=True)
        rv_sh = jnp.concatenate([rv[:, :1], rv[:, :K - 1]], axis=1)
        ri_sh = jnp.concatenate([ri[:, :1], ri[:, :K - 1]], axis=1)
        ins_v = jnp.where(kpos < p, rv, jnp.where(kpos == p, cm, rv_sh))
        ins_i = jnp.where(kpos < p, ri, jnp.where(kpos == p, gi, ri_sh))
        rv = jnp.where(do, ins_v, rv)
        ri = jnp.where(do, ins_i, ri)
        cm2, ci2 = calc_cand(cm, ci)
        return cm2, ci2, rv, ri

    _, _, rv_f, ri_f = lax.while_loop(cond, body, (cm0, ci0, rv0, ri0))
    rv_ref[...] = rv_f
    ri_ref[...] = ri_f

    @pl.when(i == nb - 1)
    def _fin():
        e = jnp.exp(rv_f - rv_f[:, :1])
        wt_out[...] = e / jnp.sum(e, axis=1, keepdims=True)
        idx_out[...] = ri_f


def _scan(qn, keys, d):
    nchunk = (MEM + CHUNK - 1) // CHUNK
    return pl.pallas_call(
        _scan_body,
        grid=(nchunk,),
        in_specs=[
            pl.BlockSpec((B, D), lambda i: (0, 0)),
            pl.BlockSpec((CHUNK, D), lambda i: (i, 0)),
            pl.BlockSpec((CHUNK, 1), lambda i: (i, 0)),
        ],
        out_specs=[
            pl.BlockSpec((B, K), lambda i: (0, 0)),
            pl.BlockSpec((B, K), lambda i: (0, 0)),
        ],
        out_shape=[
            jax.ShapeDtypeStruct((B, K), jnp.int32),
            jax.ShapeDtypeStruct((B, K), jnp.float32),
        ],
        scratch_shapes=[
            pltpu.VMEM((B, K), jnp.float32),
            pltpu.VMEM((B, K), jnp.int32),
        ],
    )(qn, keys, d)


@functools.cache
def _make_sc_read():
    mesh = plsc.VectorSubcoreMesh(core_axis_name="c", subcore_axis_name="s")

    @functools.partial(
        pl.kernel,
        mesh=mesh,
        out_type=jax.ShapeDtypeStruct((B * K,), jnp.float32),
        scratch_types=[
            pltpu.VMEM((K,), jnp.int32),
            pltpu.VMEM((K, 128), jnp.float32),
            pltpu.VMEM((K, 128), jnp.float32),
            pltpu.VMEM((D,), jnp.float32),
            pltpu.SemaphoreType.DMA,
        ],
    )
    def _sc_read(values_hbm, idx_hbm, w_hbm, out_hbm,
                 idx_v, rows_v, w_v, acc_v, sem):
        wid = lax.axis_index("s") * 2 + lax.axis_index("c")
        base = wid * K
        pltpu.sync_copy(idx_hbm.at[pl.ds(base, K)], idx_v)
        pltpu.sync_copy(w_hbm.at[pl.ds(base, K)], w_v)
        pltpu.async_copy(values_hbm.at[idx_v], rows_v, sem).wait()
        acc = [jnp.zeros((16,), jnp.float32) for _ in range(8)]
        for k in range(K):
            for s in range(8):
                acc[s] = acc[s] + (rows_v[k, 16 * s:16 * (s + 1)]
                                   * w_v[k, 16 * s:16 * (s + 1)])
        acc_v[0:16] = acc[0] + acc[2] + acc[4] + acc[6]
        acc_v[16:32] = acc[1] + acc[3] + acc[5] + acc[7]
        pltpu.sync_copy(acc_v, out_hbm.at[pl.ds(wid * D, D)])

    return _sc_read


def kernel(x, W, b, mem_keys, mem_values):
    q = x @ W.T + b
    qn = q / (jnp.linalg.norm(q, axis=-1, keepdims=True) + EPS)
    d = jnp.linalg.norm(mem_keys, axis=-1, keepdims=True) + EPS
    idx, wt = _scan(qn, mem_keys, d)
    idx_flat = idx.reshape(B * K)
    row_idx = idx_flat // 4
    grp = jnp.arange(128, dtype=jnp.int32) // D
    wmask = jnp.where(grp[None, :] == (idx_flat % 4)[:, None],
                      wt.reshape(B * K, 1), 0.0)
    values128 = mem_values.reshape(MEM // 4, 128)
    out_flat = _make_sc_read()(values128, row_idx, wmask)
    return out_flat.reshape(B, D)
